# baseline (speedup 1.0000x reference)
import functools

import jax
import jax.numpy as jnp
from jax import lax
from jax.experimental import pallas as pl
from jax.experimental.pallas import tpu as pltpu
from jax.experimental.pallas import tpu_sc as plsc

_N = 10000
_E = 320000
_D = 128
_NC = 2
_NS = 16
_NW = _NC * _NS
_EPT = _E // _NW
_K = 80
_KH = _K // 2
_NCHUNK = _EPT // _K
_ZC = 80
_NZCH = _N // _ZC


def _sc_aggregate_body(ei_hbm, w_hbm, x_hbm, out_hbm,
                       r0, r1, src_v, dst_v, w_v,
                       acc, g0, g1, ss0, ss1):
    c = lax.axis_index("c")
    s = lax.axis_index("s")
    wid = c * _NS + s
    rows = [r0, r1]
    gsem = [g0, g1]
    ssem = [ss0, ss1]

    pltpu.sync_copy(ei_hbm.at[1, wid], src_v)
    pltpu.sync_copy(ei_hbm.at[0, wid], dst_v)
    pltpu.sync_copy(w_hbm.at[wid], w_v)

    def gather(ci, g):
        for h in range(2):
            pltpu.async_copy(
                x_hbm.at[src_v.at[pl.ds(ci * _K + h * _KH, _KH)]],
                rows[g].at[pl.ds(h * _KH, _KH)], gsem[g])

    def wait_gather(g):
        for h in range(2):
            pltpu.make_async_copy(x_hbm.at[pl.ds(0, _KH)],
                                  rows[g].at[pl.ds(h * _KH, _KH)],
                                  gsem[g]).wait()

    def scatter(ci, p):
        for h in range(2):
            pltpu.async_copy(
                rows[p].at[pl.ds(h * _KH, _KH)],
                acc.at[dst_v.at[pl.ds(ci * _K + h * _KH, _KH)]],
                ssem[p], add=True)

    def wait_scatter(p):
        for h in range(2):
            pltpu.make_async_copy(rows[p].at[pl.ds(h * _KH, _KH)],
                                  acc.at[pl.ds(0, _KH)], ssem[p]).wait()

    def scale(ci, p):
        def scale_g(g, c2):
            wvec = w_v[pl.ds(ci * _K + g * 16, 16)]
            for l in range(16):
                w = wvec[l]
                e = g * 16 + l
                for j in range(_D // 16):
                    sl = pl.ds(j * 16, 16)
                    rows[p][e, sl] = rows[p][e, sl] * w
            return c2

        lax.fori_loop(0, _K // 16, scale_g, 0)

    zf = jnp.zeros((16,), jnp.float32)

    def zb(e, carry):
        for buf in rows:
            for j in range(_D // 16):
                buf[e, pl.ds(j * 16, 16)] = zf
        return carry

    lax.fori_loop(0, _ZC, zb, 0)

    nmine = jnp.where(s < _NZCH - (_NZCH // _NS) * _NS, _NZCH // _NS + 1,
                      _NZCH // _NS)

    def zloop(k, carry):
        i = k * _NS + s
        pltpu.sync_copy(rows[0], acc.at[pl.ds(i * _ZC, _ZC)])
        return carry

    lax.fori_loop(0, nmine, zloop, 0)
    plsc.subcore_barrier()

    scatter(0, 1)
    gather(0, 0)

    def round_body(r, carry):
        for k in range(2):
            i = r * 2 + k
            p = k
            o = (k + 1) % 2
            wait_gather(p)
            wait_scatter(o)
            gather(i + 1, o)
            scale(i, p)
            scatter(i, p)
        return carry

    lax.fori_loop(0, (_NCHUNK - 1) // 2, round_body, 0)

    wait_gather(0)
    wait_scatter(1)
    scale(_NCHUNK - 1, 0)
    scatter(_NCHUNK - 1, 0)
    wait_scatter(0)
    plsc.subcore_barrier()

    def wloop(k, carry):
        i = k * _NS + s
        pltpu.sync_copy(acc.at[pl.ds(i * _ZC, _ZC)],
                        out_hbm.at[c, pl.ds(i * _ZC, _ZC)])
        return carry

    lax.fori_loop(0, nmine, wloop, 0)


_sc_aggregate = functools.partial(
    pl.kernel,
    mesh=plsc.VectorSubcoreMesh(core_axis_name="c", subcore_axis_name="s"),
    out_type=jax.ShapeDtypeStruct((_NC, _N, _D), jnp.float32),
    scratch_types=(
        [pltpu.VMEM((_K, _D), jnp.float32) for _ in range(2)]
        + [pltpu.VMEM((_EPT,), jnp.int32)]
        + [pltpu.VMEM((_EPT,), jnp.int32)]
        + [pltpu.VMEM((_EPT,), jnp.float32)]
        + [pltpu.VMEM_SHARED((_N, _D), jnp.float32)]
        + [pltpu.SemaphoreType.DMA for _ in range(4)]
    ),
)(_sc_aggregate_body)


_BN = 2000


def _tc_matmul_body(p_ref, w_ref, b_ref, o_ref):
    p = p_ref[0] + p_ref[1]
    o_ref[...] = (
        jnp.dot(p, w_ref[...], preferred_element_type=jnp.float32) + b_ref[...]
    )


def _tc_matmul(partials, W, b):
    return pl.pallas_call(
        _tc_matmul_body,
        grid=(_N // _BN,),
        in_specs=[
            pl.BlockSpec((_NC, _BN, _D), lambda i: (0, i, 0)),
            pl.BlockSpec((_D, _D), lambda i: (0, 0)),
            pl.BlockSpec((1, _D), lambda i: (0, 0)),
        ],
        out_specs=pl.BlockSpec((_BN, _D), lambda i: (i, 0)),
        out_shape=jax.ShapeDtypeStruct((_N, _D), jnp.float32),
    )(partials, W, b.reshape(1, _D))


def kernel(input, edge_index, edge_weight, W, b):
    ei = edge_index.astype(jnp.int32).reshape(2, _NW, _EPT)
    w2 = edge_weight.astype(jnp.float32).reshape(_NW, _EPT)
    partials = _sc_aggregate(ei, w2, input)
    return _tc_matmul(partials, W, b)
